# baseline (speedup 1.0000x reference)
import functools

import jax
import jax.numpy as jnp
import numpy as np
from jax import lax
from jax.experimental import pallas as pl
from jax.experimental.pallas import tpu as pltpu
from jax.experimental.pallas import tpu_sc as plsc

VOCAB = 205
D = 16
CHUNK = 1024
NB = 4
NC, NS = 2, 16
NW = NC * NS
UNROLL = 4
SLOT = 2 * 8 * 8 * 128

_LANE = np.arange(16, dtype=np.int32)
_DV = [(_LANE + k) & 15 for k in range(16)]
_SVEC = [((dv >> 3) * 8192 + (dv & 7) * 128 + _LANE).astype(np.int32)
         for dv in _DV]


def _build(B):
    assert B % (NW * CHUNK) == 0
    per_w = B // NW
    nchunks = per_w // CHUNK
    assert nchunks % NB == 0
    nrounds = nchunks // NB
    assert nrounds >= 3
    half = (B // 128) * 1024

    mesh = plsc.VectorSubcoreMesh(core_axis_name="c", subcore_axis_name="s")

    @functools.partial(
        pl.kernel,
        out_type=jax.ShapeDtypeStruct((B * D,), jnp.float32),
        mesh=mesh,
        scratch_types=(
            [pltpu.VMEM((NB * CHUNK,), jnp.int32),
             pltpu.VMEM((NB * SLOT,), jnp.float32),
             pltpu.VMEM((16 * VOCAB * D,), jnp.float32)]
            + [pltpu.SemaphoreType.DMA] * NB
            + [pltpu.SemaphoreType.DMA] * NB
        ),
        compiler_params=pltpu.CompilerParams(
            use_tc_tiling_on_sc=False, needs_layout_passes=False),
    )
    def k(x_hbm, table_hbm, out_hbm, idx_v, rows_v, tbl_v, *sems):
        sem_i = sems[:NB]
        sem_w = sems[NB:]
        wid = lax.axis_index("s") * NC + lax.axis_index("c")
        j0 = wid * per_w

        pltpu.sync_copy(table_hbm, tbl_v)

        lane = lax.iota(jnp.int32, 16)
        dvs = [(lane + kk) & 15 for kk in range(16)]
        svecs = [((dv >> 3) * 8192 + (dv & 7) * 128 + lane) for dv in dvs]
        tbls = [tbl_v.at[pl.ds(kk * VOCAB * D, VOCAB * D)] for kk in range(16)]

        def idx_load(g, b):
            return pltpu.make_async_copy(
                x_hbm.at[pl.ds(j0 + g * CHUNK, CHUNK)],
                idx_v.at[pl.ds(b * CHUNK, CHUNK)], sem_i[b])

        def write(g, b, t):
            dst = t * half + (j0 + g * CHUNK) * 8
            return pltpu.make_async_copy(
                rows_v.at[pl.ds(b * SLOT + t * 8192, 8192)],
                out_hbm.at[pl.ds(dst, 8192)], sem_w[b])

        def compute(b):
            def body(p, _):
                gaddr = idx_v[pl.ds(b * CHUNK + p * 16, 16)] * D + lane
                sbase = b * SLOT + (p >> 3) * 1024 + (p & 7) * 16
                for kk in range(16):
                    vals = plsc.load_gather(tbls[kk], [gaddr])
                    plsc.store_scatter(rows_v, [svecs[kk] + sbase], vals)
                return 0
            lax.fori_loop(0, CHUNK // 16, body, 0, unroll=UNROLL)

        for b in range(NB):
            idx_load(b, b).start()

        for b in range(NB):
            idx_load(b, b).wait()
            compute(b)
            write(b, b, 0).start()
            write(b, b, 1).start()
            idx_load(b + NB, b).start()

        def round_body(r, _):
            for b in range(NB):
                g = r * NB + b
                idx_load(g, b).wait()
                write(g - NB, b, 0).wait()
                write(g - NB, b, 1).wait()
                compute(b)
                write(g, b, 0).start()
                write(g, b, 1).start()
                idx_load(g + NB, b).start()
            return 0

        lax.fori_loop(1, nrounds - 1, round_body, 0)

        r = nrounds - 1
        for b in range(NB):
            g = r * NB + b
            idx_load(g, b).wait()
            write(g - NB, b, 0).wait()
            write(g - NB, b, 1).wait()
            compute(b)
            write(g, b, 0).start()
            write(g, b, 1).start()
        for b in range(NB):
            write(r * NB + b, b, 0).wait()
            write(r * NB + b, b, 1).wait()

    return k


def kernel(x, embed_site):
    B = x.size
    cols = (np.arange(D)[None, :] + np.arange(D)[:, None]) % D
    tbl16 = embed_site[:, cols].transpose(1, 0, 2).reshape(-1)
    flat = _build(B)(x.reshape(B).astype(jnp.int32), tbl16)
    return flat.reshape(2, B // 128, 8, 128).transpose(1, 3, 0, 2).reshape(B, D)


# speedup vs baseline: 2.2135x; 2.2135x over previous
import functools

import jax
import jax.numpy as jnp
import numpy as np
from jax import lax
from jax.experimental import pallas as pl
from jax.experimental.pallas import tpu as pltpu
from jax.experimental.pallas import tpu_sc as plsc

VOCAB = 205
D = 16
CHUNK = 1024
NB = 4
NC, NS = 2, 16
NW = NC * NS
UNROLL = 4
SLOT = 2 * 8 * 8 * 128

_LANE = np.arange(16, dtype=np.int32)
_DV = [(_LANE + k) & 15 for k in range(16)]
_SVEC = [((dv >> 3) * 8192 + (dv & 7) * 128 + _LANE).astype(np.int32)
         for dv in _DV]


def _build(B):
    assert B % (NW * CHUNK) == 0
    per_w = B // NW
    nchunks = per_w // CHUNK
    assert nchunks % NB == 0
    nrounds = nchunks // NB
    assert nrounds >= 3
    half = (B // 128) * 1024

    mesh = plsc.VectorSubcoreMesh(core_axis_name="c", subcore_axis_name="s")

    @functools.partial(
        pl.kernel,
        out_type=jax.ShapeDtypeStruct((B * D,), jnp.float32),
        mesh=mesh,
        scratch_types=(
            [pltpu.VMEM((NB * CHUNK,), jnp.int32),
             pltpu.VMEM((NB * SLOT,), jnp.float32),
             pltpu.VMEM((16 * VOCAB * D,), jnp.float32)]
            + [pltpu.SemaphoreType.DMA] * NB
            + [pltpu.SemaphoreType.DMA] * NB
        ),
        compiler_params=pltpu.CompilerParams(
            use_tc_tiling_on_sc=False, needs_layout_passes=False),
    )
    def k(x_hbm, table_hbm, out_hbm, idx_v, rows_v, tbl_v, *sems):
        sem_i = sems[:NB]
        sem_w = sems[NB:]
        wid = lax.axis_index("s") * NC + lax.axis_index("c")
        j0 = wid * per_w

        pltpu.sync_copy(table_hbm, tbl_v)

        lane = lax.iota(jnp.int32, 16)
        dvs = [(lane + kk) & 15 for kk in range(16)]
        svecs = [((dv >> 3) * 8192 + (dv & 7) * 128 + lane) for dv in dvs]
        tbls = [tbl_v.at[pl.ds(kk * VOCAB * D, VOCAB * D)] for kk in range(16)]

        def idx_load(g, b):
            return pltpu.make_async_copy(
                x_hbm.at[pl.ds(j0 + g * CHUNK, CHUNK)],
                idx_v.at[pl.ds(b * CHUNK, CHUNK)], sem_i[b])

        def write(g, b, t):
            dst = t * half + (j0 + g * CHUNK) * 8
            return pltpu.make_async_copy(
                rows_v.at[pl.ds(b * SLOT + t * 8192, 8192)],
                out_hbm.at[pl.ds(dst, 8192)], sem_w[b])

        def compute(b):
            @plsc.parallel_loop(0, CHUNK // 16, unroll=UNROLL)
            def body(p):
                gaddr = idx_v[pl.ds(b * CHUNK + p * 16, 16)] * D + lane
                sbase = b * SLOT + (p >> 3) * 1024 + (p & 7) * 16
                for kk in range(16):
                    vals = plsc.load_gather(tbls[kk], [gaddr])
                    plsc.store_scatter(rows_v, [svecs[kk] + sbase], vals)

        for b in range(NB):
            idx_load(b, b).start()

        for b in range(NB):
            idx_load(b, b).wait()
            compute(b)
            write(b, b, 0).start()
            write(b, b, 1).start()
            idx_load(b + NB, b).start()

        def round_body(r, _):
            for b in range(NB):
                g = r * NB + b
                idx_load(g, b).wait()
                write(g - NB, b, 0).wait()
                write(g - NB, b, 1).wait()
                compute(b)
                write(g, b, 0).start()
                write(g, b, 1).start()
                idx_load(g + NB, b).start()
            return 0

        lax.fori_loop(1, nrounds - 1, round_body, 0)

        r = nrounds - 1
        for b in range(NB):
            g = r * NB + b
            idx_load(g, b).wait()
            write(g - NB, b, 0).wait()
            write(g - NB, b, 1).wait()
            compute(b)
            write(g, b, 0).start()
            write(g, b, 1).start()
        for b in range(NB):
            write(r * NB + b, b, 0).wait()
            write(r * NB + b, b, 1).wait()

    return k


def kernel(x, embed_site):
    B = x.size
    cols = (np.arange(D)[None, :] + np.arange(D)[:, None]) % D
    tbl16 = embed_site[:, cols].transpose(1, 0, 2).reshape(-1)
    flat = _build(B)(x.reshape(B).astype(jnp.int32), tbl16)
    return flat.reshape(2, B // 128, 8, 128).transpose(1, 3, 0, 2).reshape(B, D)


# speedup vs baseline: 3.2071x; 1.4489x over previous
import functools

import jax
import jax.numpy as jnp
import numpy as np
from jax import lax
from jax.experimental import pallas as pl
from jax.experimental.pallas import tpu as pltpu
from jax.experimental.pallas import tpu_sc as plsc

VOCAB = 205
D = 16
CHUNK = 2048
NB = 2
NC, NS = 2, 16
NW = NC * NS
UNROLL = 4
SLOT = 2 * 16 * 8 * 128

_LANE = np.arange(16, dtype=np.int32)
_DV = [(_LANE + k) & 15 for k in range(16)]
_SVEC = [((dv >> 3) * 8192 + (dv & 7) * 128 + _LANE).astype(np.int32)
         for dv in _DV]


def _build(B):
    assert B % (NW * CHUNK) == 0
    per_w = B // NW
    nchunks = per_w // CHUNK
    assert nchunks % NB == 0
    nrounds = nchunks // NB
    assert nrounds >= 3
    half = (B // 128) * 1024

    mesh = plsc.VectorSubcoreMesh(core_axis_name="c", subcore_axis_name="s")

    @functools.partial(
        pl.kernel,
        out_type=jax.ShapeDtypeStruct((B * D,), jnp.float32),
        mesh=mesh,
        scratch_types=(
            [pltpu.VMEM((NB * CHUNK,), jnp.int32),
             pltpu.VMEM((NB * SLOT,), jnp.float32),
             pltpu.VMEM((VOCAB * D,), jnp.float32)]
            + [pltpu.SemaphoreType.DMA] * NB
            + [pltpu.SemaphoreType.DMA] * NB
        ),
        compiler_params=pltpu.CompilerParams(
            use_tc_tiling_on_sc=False, needs_layout_passes=False),
    )
    def k(x_hbm, table_hbm, out_hbm, idx_v, rows_v, tbl_v, *sems):
        sem_i = sems[:NB]
        sem_w = sems[NB:]
        wid = lax.axis_index("s") * NC + lax.axis_index("c")
        j0 = wid * per_w

        pltpu.sync_copy(table_hbm, tbl_v)

        lane = lax.iota(jnp.int32, 16)
        dvs = [(lane + kk) & 15 for kk in range(16)]
        svecs = [((dv >> 3) * (SLOT // 2) + (dv & 7) * 128 + lane) for dv in dvs]

        def idx_load(g, b):
            return pltpu.make_async_copy(
                x_hbm.at[pl.ds(j0 + g * CHUNK, CHUNK)],
                idx_v.at[pl.ds(b * CHUNK, CHUNK)], sem_i[b])

        def write(g, b, t):
            dst = t * half + (j0 + g * CHUNK) * 8
            return pltpu.make_async_copy(
                rows_v.at[pl.ds(b * SLOT + t * (SLOT // 2), SLOT // 2)],
                out_hbm.at[pl.ds(dst, SLOT // 2)], sem_w[b])

        def compute(b):
            @plsc.parallel_loop(0, CHUNK // 16, unroll=UNROLL)
            def body(p):
                iv = idx_v[pl.ds(b * CHUNK + p * 16, 16)] * D
                sbase = b * SLOT + (p >> 3) * 1024 + (p & 7) * 16
                for kk in range(16):
                    vals = plsc.load_gather(tbl_v, [iv + dvs[kk]])
                    plsc.store_scatter(rows_v, [svecs[kk] + sbase], vals)

        for b in range(NB):
            idx_load(b, b).start()

        for b in range(NB):
            idx_load(b, b).wait()
            compute(b)
            write(b, b, 0).start()
            write(b, b, 1).start()
            idx_load(b + NB, b).start()

        def round_body(r, _):
            for b in range(NB):
                g = r * NB + b
                idx_load(g, b).wait()
                write(g - NB, b, 0).wait()
                write(g - NB, b, 1).wait()
                compute(b)
                write(g, b, 0).start()
                write(g, b, 1).start()
                idx_load(g + NB, b).start()
            return 0

        lax.fori_loop(1, nrounds - 1, round_body, 0)

        r = nrounds - 1
        for b in range(NB):
            g = r * NB + b
            idx_load(g, b).wait()
            write(g - NB, b, 0).wait()
            write(g - NB, b, 1).wait()
            compute(b)
            write(g, b, 0).start()
            write(g, b, 1).start()
        for b in range(NB):
            write(r * NB + b, b, 0).wait()
            write(r * NB + b, b, 1).wait()

    return k


def kernel(x, embed_site):
    B = x.size
    flat = _build(B)(x.reshape(B).astype(jnp.int32), embed_site.reshape(-1))
    return flat.reshape(2, B // 128, 8, 128).transpose(1, 3, 0, 2).reshape(B, D)
